# Initial kernel scaffold; baseline (speedup 1.0000x reference)
import jax
import jax.numpy as jnp
from jax.experimental import pallas as pl


def kernel(node_embeddings, node_embeddings_bias, W1, b1, W2, b2, relations, graph, triples):
    raise NotImplementedError("write your pallas kernel here")


# speedup vs baseline: 5.7043x; 5.7043x over previous
import functools

import jax
import jax.numpy as jnp
from jax import lax
from jax.experimental import pallas as pl
from jax.experimental.pallas import tpu as pltpu
from jax.experimental.pallas import tpu_sc as plsc

NNODES = 10000
NREL = 16
NEMB = 64
R_TOTAL = 2 * NREL + 1
SEGTOT = R_TOTAL * NNODES

NC, NS, L = 2, 16, 16
NW = NC * NS

E_REAL = 2 * 320000 + NNODES
CH = 128
NCHUNK = -(-E_REAL // (NW * CH))
PER_TILE = NCHUNK * CH
EP = NW * PER_TILE

SEG_FLAT = 331776
SROWS = SEG_FLAT // 16
DEAD_SEG = SEGTOT

NB = 10
BM = NNODES // NB

N_TRIPLES = 16384
T_PER_TILE = N_TRIPLES // NW
T_CHUNKS = T_PER_TILE // CH

_MESH = plsc.VectorSubcoreMesh(core_axis_name="c", subcore_axis_name="s")


def _iota16():
    return lax.iota(jnp.int32, L)


def _z1_body(x_ref, b_ref, w_ref, z_ref):
    xb = jax.nn.relu(x_ref[...] + b_ref[...])
    z_ref[0] = jnp.dot(xb, w_ref[0], preferred_element_type=jnp.float32)


def _z1_call(x, bias, W):
    return pl.pallas_call(
        _z1_body,
        grid=(NB, R_TOTAL),
        in_specs=[
            pl.BlockSpec((BM, NEMB), lambda i, r: (i, 0)),
            pl.BlockSpec((1, NEMB), lambda i, r: (0, 0)),
            pl.BlockSpec((1, NEMB, NEMB), lambda i, r: (r, 0, 0)),
        ],
        out_specs=pl.BlockSpec((1, BM, NEMB), lambda i, r: (r, i, 0)),
        out_shape=jax.ShapeDtypeStruct((R_TOTAL, NNODES, NEMB), jnp.float32),
    )(x, bias, W)


def _z2_body(p_ref, b_ref, w_ref, z_ref):
    xb = jax.nn.relu(p_ref[0] + p_ref[1] + b_ref[...])
    z_ref[0] = jnp.dot(xb, w_ref[0], preferred_element_type=jnp.float32)


def _z2_call(p, b1, W):
    return pl.pallas_call(
        _z2_body,
        grid=(NB, R_TOTAL),
        in_specs=[
            pl.BlockSpec((NC, BM, NEMB), lambda i, r: (0, i, 0)),
            pl.BlockSpec((1, NEMB), lambda i, r: (0, 0)),
            pl.BlockSpec((1, NEMB, NEMB), lambda i, r: (r, 0, 0)),
        ],
        out_specs=pl.BlockSpec((1, BM, NEMB), lambda i, r: (r, i, 0)),
        out_shape=jax.ShapeDtypeStruct((R_TOTAL, NNODES, NEMB), jnp.float32),
    )(p, b1, W)


def _counts_body(seg_hbm, out_hbm, segv, onesv, zbuf, acc, sem):
    del sem
    cid = lax.axis_index("c")
    sid = lax.axis_index("s")
    wid = sid * NC + cid
    base = wid * PER_TILE
    seg_per_tile = SEG_FLAT // NS
    zsize = seg_per_tile // 8

    @pl.loop(0, zsize // L)
    def _zero(i):
        zbuf[pl.ds(i * L, L)] = jnp.zeros((L,), jnp.float32)

    @pl.loop(0, 8)
    def _zacc(k):
        pltpu.sync_copy(zbuf, acc.at[pl.ds(sid * seg_per_tile + k * zsize, zsize)])

    @pl.loop(0, CH // L)
    def _ones(g):
        onesv[pl.ds(g * L, L)] = jnp.ones((L,), jnp.float32)

    plsc.subcore_barrier()

    @pl.loop(0, NCHUNK)
    def _chunk(t):
        off = base + t * CH
        pltpu.sync_copy(seg_hbm.at[pl.ds(off, CH)], segv)
        pltpu.sync_copy(onesv, acc.at[segv], add=True)

    plsc.subcore_barrier()
    pltpu.sync_copy(
        acc.at[pl.ds(sid * seg_per_tile, seg_per_tile)],
        out_hbm.at[pl.ds(cid * SEG_FLAT + sid * seg_per_tile, seg_per_tile)],
    )


def _counts_call(seg):
    return pl.kernel(
        _counts_body,
        out_type=jax.ShapeDtypeStruct((NC * SEG_FLAT,), jnp.float32),
        mesh=_MESH,
        scratch_types=[
            pltpu.VMEM((CH,), jnp.int32),
            pltpu.VMEM((CH,), jnp.float32),
            pltpu.VMEM((SEG_FLAT // NS // 8,), jnp.float32),
            pltpu.VMEM_SHARED((SEG_FLAT,), jnp.float32),
            pltpu.SemaphoreType.DMA,
        ],
    )(seg)


def _inv_body(c_ref, inv_ref):
    c = c_ref[0] + c_ref[1]
    rows, cols = c.shape
    flat = (
        lax.broadcasted_iota(jnp.int32, (rows, cols), 0) * cols
        + lax.broadcasted_iota(jnp.int32, (rows, cols), 1)
    )
    inv = 1.0 / jnp.maximum(c, 1.0)
    inv_ref[...] = jnp.where(flat < SEGTOT, inv, 0.0)


def _inv_call(counts2):
    c = counts2.reshape(NC, SEG_FLAT // 128, 128)
    return pl.pallas_call(
        _inv_body,
        out_shape=jax.ShapeDtypeStruct((SEG_FLAT // 128, 128), jnp.float32),
    )(c)


def _norms_body(seg_hbm, inv_hbm, out_hbm, segv, normv, sem):
    cid = lax.axis_index("c")
    sid = lax.axis_index("s")
    wid = sid * NC + cid
    base = wid * PER_TILE

    @pl.loop(0, NCHUNK)
    def _chunk(t):
        off = base + t * CH
        pltpu.sync_copy(seg_hbm.at[pl.ds(off, CH)], segv)
        pltpu.async_copy(inv_hbm.at[segv], normv, sem).wait()
        pltpu.sync_copy(normv, out_hbm.at[pl.ds(off, CH)])


def _norms_call(seg, inv):
    invf = inv.reshape(SEG_FLAT)
    return pl.kernel(
        _norms_body,
        out_type=jax.ShapeDtypeStruct((EP,), jnp.float32),
        mesh=_MESH,
        scratch_types=[
            pltpu.VMEM((CH,), jnp.int32),
            pltpu.VMEM((CH,), jnp.float32),
            pltpu.SemaphoreType.DMA,
        ],
    )(seg, invf)


def _layer_body(z_hbm, gidx_hbm, oidx_hbm, norms_hbm, out_hbm,
                gv, ov, nv, rows, zbuf, acc, sem):
    cid = lax.axis_index("c")
    sid = lax.axis_index("s")
    wid = sid * NC + cid
    base = wid * PER_TILE
    nrows = 624
    zrows = 208

    @pl.loop(0, zrows)
    def _zero(i):
        for j in range(NEMB // L):
            zbuf[i, pl.ds(j * L, L)] = jnp.zeros((L,), jnp.float32)

    @pl.loop(0, nrows // zrows)
    def _zacc(k):
        pltpu.sync_copy(zbuf, acc.at[pl.ds(sid * nrows + k * zrows, zrows)])

    @pl.when(sid == NS - 1)
    def _ztail():
        pltpu.sync_copy(zbuf.at[pl.ds(0, 16)], acc.at[pl.ds(NS * nrows, 16)])

    plsc.subcore_barrier()

    @pl.loop(0, NCHUNK)
    def _chunk(t):
        off = base + t * CH
        pltpu.sync_copy(gidx_hbm.at[pl.ds(off, CH)], gv)
        pltpu.sync_copy(oidx_hbm.at[pl.ds(off, CH)], ov)
        pltpu.sync_copy(norms_hbm.at[pl.ds(off, CH)], nv)
        pltpu.async_copy(z_hbm.at[gv], rows, sem).wait()
        for g in range(CH // L):
            nvec = nv[pl.ds(g * L, L)]
            for i in range(L):
                nbs = nvec[i]
                e = g * L + i
                for j in range(NEMB // L):
                    sl = pl.ds(j * L, L)
                    rows[e, sl] = rows[e, sl] * nbs
        pltpu.sync_copy(rows, acc.at[ov], add=True)

    plsc.subcore_barrier()
    pltpu.sync_copy(
        acc.at[pl.ds(sid * nrows, nrows)],
        out_hbm.at[cid, pl.ds(sid * nrows, nrows)],
    )

    @pl.when(sid == NS - 1)
    def _etail():
        pltpu.sync_copy(
            acc.at[pl.ds(NS * nrows, 16)],
            out_hbm.at[cid, pl.ds(NS * nrows, 16)],
        )


def _layer_call(z, gidx, oidx, norms):
    zflat = z.reshape(R_TOTAL * NNODES, NEMB)
    return pl.kernel(
        _layer_body,
        out_type=jax.ShapeDtypeStruct((NC, NNODES, NEMB), jnp.float32),
        mesh=_MESH,
        compiler_params=pltpu.CompilerParams(
            needs_layout_passes=False, use_tc_tiling_on_sc=False),
        scratch_types=[
            pltpu.VMEM((CH,), jnp.int32),
            pltpu.VMEM((CH,), jnp.int32),
            pltpu.VMEM((CH,), jnp.float32),
            pltpu.VMEM((CH, NEMB), jnp.float32),
            pltpu.VMEM((208, NEMB), jnp.float32),
            pltpu.VMEM_SHARED((NNODES, NEMB), jnp.float32),
            pltpu.SemaphoreType.DMA,
        ],
    )(zflat, gidx, oidx, norms)


def _combine_body(p_ref, b_ref, rel_ref, x_ref, pen_ref):
    x_ref[...] = p_ref[0] + p_ref[1] + b_ref[...]

    @pl.when(pl.program_id(0) == 0)
    def _():
        pen_ref[...] = jnp.sum(rel_ref[...] ** 2).reshape(1, 1)


def _combine_call(p, b2, relations):
    return pl.pallas_call(
        _combine_body,
        grid=(NB,),
        in_specs=[
            pl.BlockSpec((NC, BM, NEMB), lambda i: (0, i, 0)),
            pl.BlockSpec((1, NEMB), lambda i: (0, 0)),
            pl.BlockSpec((NREL, NEMB), lambda i: (0, 0)),
        ],
        out_specs=[
            pl.BlockSpec((BM, NEMB), lambda i: (i, 0)),
            pl.BlockSpec((1, 1), lambda i: (0, 0)),
        ],
        out_shape=[
            jax.ShapeDtypeStruct((NNODES, NEMB), jnp.float32),
            jax.ShapeDtypeStruct((1, 1), jnp.float32),
        ],
    )(p, b2, relations)


def _tgather_body(x_hbm, rel_hbm, ts_hbm, tp_hbm, to_hbm, out_hbm,
                  tsv, tpv, tov, A, B, C, sem):
    cid = lax.axis_index("c")
    sid = lax.axis_index("s")
    wid = sid * NC + cid
    base = wid * T_PER_TILE

    @pl.loop(0, T_CHUNKS)
    def _chunk(t):
        off = base + t * CH
        pltpu.sync_copy(ts_hbm.at[pl.ds(off, CH)], tsv)
        pltpu.sync_copy(tp_hbm.at[pl.ds(off, CH)], tpv)
        pltpu.sync_copy(to_hbm.at[pl.ds(off, CH)], tov)
        pltpu.async_copy(x_hbm.at[tsv], A, sem).wait()
        pltpu.async_copy(x_hbm.at[tov], B, sem).wait()
        pltpu.async_copy(rel_hbm.at[tpv], C, sem).wait()
        pltpu.sync_copy(A, out_hbm.at[0, pl.ds(off, CH)])
        pltpu.sync_copy(B, out_hbm.at[1, pl.ds(off, CH)])
        pltpu.sync_copy(C, out_hbm.at[2, pl.ds(off, CH)])


def _tgather_call(x2, relations, ts, tp, to):
    return pl.kernel(
        _tgather_body,
        out_type=jax.ShapeDtypeStruct((3, N_TRIPLES, NEMB), jnp.float32),
        mesh=_MESH,
        compiler_params=pltpu.CompilerParams(use_tc_tiling_on_sc=False),
        scratch_types=[
            pltpu.VMEM((CH,), jnp.int32),
            pltpu.VMEM((CH,), jnp.int32),
            pltpu.VMEM((CH,), jnp.int32),
            pltpu.VMEM((CH, NEMB), jnp.float32),
            pltpu.VMEM((CH, NEMB), jnp.float32),
            pltpu.VMEM((CH, NEMB), jnp.float32),
            pltpu.SemaphoreType.DMA,
        ],
    )(x2, relations, ts, tp, to)


BT = 2048


def _scores_body(abc_ref, s_ref):
    prod = abc_ref[0] * abc_ref[1] * abc_ref[2]
    s_ref[...] = jnp.sum(prod, axis=-1, keepdims=True)


def _scores_call(abc):
    out = pl.pallas_call(
        _scores_body,
        grid=(N_TRIPLES // BT,),
        in_specs=[pl.BlockSpec((3, BT, NEMB), lambda i: (0, i, 0))],
        out_specs=pl.BlockSpec((BT, 1), lambda i: (i, 0)),
        out_shape=jax.ShapeDtypeStruct((N_TRIPLES, 1), jnp.float32),
    )(abc)
    return out.reshape(N_TRIPLES)


def kernel(node_embeddings, node_embeddings_bias, W1, b1, W2, b2, relations,
           graph, triples):
    s = graph[:, 0].astype(jnp.int32)
    r = (graph[:, 1] % NREL).astype(jnp.int32)
    o = graph[:, 2].astype(jnp.int32)
    loop = jnp.arange(NNODES, dtype=jnp.int32)
    s_aug = jnp.concatenate([s, o, loop])
    o_aug = jnp.concatenate([o, s, loop])
    r_aug = jnp.concatenate([r, r + NREL, jnp.full((NNODES,), 2 * NREL, jnp.int32)])

    pad = EP - E_REAL
    gidx = jnp.concatenate([r_aug * NNODES + s_aug, jnp.zeros((pad,), jnp.int32)])
    seg = jnp.concatenate([r_aug * NNODES + o_aug, jnp.full((pad,), DEAD_SEG, jnp.int32)])
    oidx = jnp.concatenate([o_aug, jnp.zeros((pad,), jnp.int32)])

    ts = triples[:, 0].astype(jnp.int32)
    tp = (triples[:, 1] % NREL).astype(jnp.int32)
    to = triples[:, 2].astype(jnp.int32)

    bias = node_embeddings_bias.reshape(1, NEMB)
    b1r = b1.reshape(1, NEMB)
    b2r = b2.reshape(1, NEMB)

    counts2 = _counts_call(seg)
    inv = _inv_call(counts2)
    norms = _norms_call(seg, inv)

    z1 = _z1_call(node_embeddings, bias, W1)
    p1 = _layer_call(z1, gidx, oidx, norms)

    z2 = _z2_call(p1, b1r, W2)
    p2 = _layer_call(z2, gidx, oidx, norms)

    x2, pen = _combine_call(p2, b2r, relations)
    abc = _tgather_call(x2, relations, ts, tp, to)
    scores = _scores_call(abc)
    return (scores, pen.reshape(()))


# speedup vs baseline: 9.4429x; 1.6554x over previous
import functools

import jax
import jax.numpy as jnp
from jax import lax
from jax.experimental import pallas as pl
from jax.experimental.pallas import tpu as pltpu
from jax.experimental.pallas import tpu_sc as plsc

NNODES = 10000
NREL = 16
NEMB = 64
R_TOTAL = 2 * NREL + 1
SEGTOT = R_TOTAL * NNODES

NC, NS, L = 2, 16, 16
NW = NC * NS

E_REAL = 2 * 320000 + NNODES
CH = 128
SUP = 32
NCHUNK = 160
NSUPER = NCHUNK // SUP
PER_TILE = NCHUNK * CH
EP = NW * PER_TILE

SEG_FLAT = 331776
SROWS = SEG_FLAT // 16
DEAD_SEG = SEGTOT

NB = 10
BM = NNODES // NB

N_TRIPLES = 16384
T_PER_TILE = N_TRIPLES // NW
T_CHUNKS = T_PER_TILE // CH

_MESH = plsc.VectorSubcoreMesh(core_axis_name="c", subcore_axis_name="s")


def _iota16():
    return lax.iota(jnp.int32, L)


def _z1_body(x_ref, b_ref, w_ref, z_ref):
    xb = jax.nn.relu(x_ref[...] + b_ref[...])
    z_ref[...] = jnp.dot(xb, w_ref[...], preferred_element_type=jnp.float32)


def _z1_call(x, bias, Wcat):
    return pl.pallas_call(
        _z1_body,
        grid=(NB,),
        in_specs=[
            pl.BlockSpec((BM, NEMB), lambda i: (i, 0)),
            pl.BlockSpec((1, NEMB), lambda i: (0, 0)),
            pl.BlockSpec((NEMB, R_TOTAL * NEMB), lambda i: (0, 0)),
        ],
        out_specs=pl.BlockSpec((BM, R_TOTAL * NEMB), lambda i: (i, 0)),
        out_shape=jax.ShapeDtypeStruct((NNODES, R_TOTAL * NEMB), jnp.float32),
    )(x, bias, Wcat)


def _z2_body(p_ref, b_ref, w_ref, z_ref):
    xb = jax.nn.relu(p_ref[0] + p_ref[1] + b_ref[...])
    z_ref[...] = jnp.dot(xb, w_ref[...], preferred_element_type=jnp.float32)


def _z2_call(p, b1, Wcat):
    return pl.pallas_call(
        _z2_body,
        grid=(NB,),
        in_specs=[
            pl.BlockSpec((NC, BM, NEMB), lambda i: (0, i, 0)),
            pl.BlockSpec((1, NEMB), lambda i: (0, 0)),
            pl.BlockSpec((NEMB, R_TOTAL * NEMB), lambda i: (0, 0)),
        ],
        out_specs=pl.BlockSpec((BM, R_TOTAL * NEMB), lambda i: (i, 0)),
        out_shape=jax.ShapeDtypeStruct((NNODES, R_TOTAL * NEMB), jnp.float32),
    )(p, b1, Wcat)


def _counts_body(seg_hbm, out_hbm, segv, onesv, zbuf, acc, sem):
    del sem
    cid = lax.axis_index("c")
    sid = lax.axis_index("s")
    wid = sid * NC + cid
    base = wid * PER_TILE
    seg_per_tile = SEG_FLAT // NS
    zsize = seg_per_tile // 8

    @pl.loop(0, zsize // L)
    def _zero(i):
        zbuf[pl.ds(i * L, L)] = jnp.zeros((L,), jnp.float32)

    @pl.loop(0, 8)
    def _zacc(k):
        pltpu.sync_copy(zbuf, acc.at[pl.ds(sid * seg_per_tile + k * zsize, zsize)])

    @pl.loop(0, CH // L)
    def _ones(g):
        onesv[pl.ds(g * L, L)] = jnp.ones((L,), jnp.float32)

    plsc.subcore_barrier()

    @pl.loop(0, NCHUNK)
    def _chunk(t):
        off = base + t * CH
        pltpu.sync_copy(seg_hbm.at[pl.ds(off, CH)], segv)
        pltpu.sync_copy(onesv, acc.at[segv], add=True)

    plsc.subcore_barrier()
    pltpu.sync_copy(
        acc.at[pl.ds(sid * seg_per_tile, seg_per_tile)],
        out_hbm.at[pl.ds(cid * SEG_FLAT + sid * seg_per_tile, seg_per_tile)],
    )


def _counts_call(seg):
    return pl.kernel(
        _counts_body,
        out_type=jax.ShapeDtypeStruct((NC * SEG_FLAT,), jnp.float32),
        mesh=_MESH,
        scratch_types=[
            pltpu.VMEM((CH,), jnp.int32),
            pltpu.VMEM((CH,), jnp.float32),
            pltpu.VMEM((SEG_FLAT // NS // 8,), jnp.float32),
            pltpu.VMEM_SHARED((SEG_FLAT,), jnp.float32),
            pltpu.SemaphoreType.DMA,
        ],
    )(seg)


def _inv_body(c_ref, inv_ref):
    c = c_ref[0] + c_ref[1]
    rows, cols = c.shape
    flat = (
        lax.broadcasted_iota(jnp.int32, (rows, cols), 0) * cols
        + lax.broadcasted_iota(jnp.int32, (rows, cols), 1)
    )
    inv = 1.0 / jnp.maximum(c, 1.0)
    inv_ref[...] = jnp.where(flat < SEGTOT, inv, 0.0)


def _inv_call(counts2):
    c = counts2.reshape(NC, SEG_FLAT // 128, 128)
    return pl.pallas_call(
        _inv_body,
        out_shape=jax.ShapeDtypeStruct((SEG_FLAT // 128, 128), jnp.float32),
    )(c)


def _norms_body(seg_hbm, inv_hbm, out_hbm, segv, normv, sem):
    cid = lax.axis_index("c")
    sid = lax.axis_index("s")
    wid = sid * NC + cid
    base = wid * PER_TILE

    @pl.loop(0, NCHUNK)
    def _chunk(t):
        off = base + t * CH
        pltpu.sync_copy(seg_hbm.at[pl.ds(off, CH)], segv)
        pltpu.async_copy(inv_hbm.at[segv], normv, sem).wait()
        pltpu.sync_copy(normv, out_hbm.at[pl.ds(off, CH)])


def _norms_call(seg, inv):
    invf = inv.reshape(SEG_FLAT)
    return pl.kernel(
        _norms_body,
        out_type=jax.ShapeDtypeStruct((EP,), jnp.float32),
        mesh=_MESH,
        scratch_types=[
            pltpu.VMEM((CH,), jnp.int32),
            pltpu.VMEM((CH,), jnp.float32),
            pltpu.SemaphoreType.DMA,
        ],
    )(seg, invf)


def _layer_body(z_hbm, gidx_hbm, oidx2_hbm, norms_hbm, out_hbm,
                gbuf, obuf, nbuf, rowsA, rowsB, zbuf, acc, gsemA, gsemB):
    cid = lax.axis_index("c")
    sid = lax.axis_index("s")
    wid = sid * NC + cid
    base = wid * PER_TILE
    nrows = 624
    zrows = 208

    @pl.loop(0, zrows)
    def _zero(i):
        for j in range(NEMB // L):
            zbuf[i, pl.ds(j * L, L)] = jnp.zeros((L,), jnp.float32)

    @pl.loop(0, nrows // zrows)
    def _zacc(k):
        pltpu.sync_copy(zbuf, acc.at[pl.ds(sid * nrows + k * zrows, zrows)])

    @pl.when(sid == NS - 1)
    def _ztail():
        pltpu.sync_copy(zbuf.at[pl.ds(0, 16)], acc.at[pl.ds(NS * nrows, 16)])

    plsc.subcore_barrier()

    def _fire(k, buf, sem):
        pltpu.async_copy(z_hbm.at[gbuf.at[pl.ds(k * CH, CH)]], buf, sem)

    def _process(k, buf, sem):
        pltpu.make_async_copy(
            z_hbm.at[gbuf.at[pl.ds(k * CH, CH)]], buf, sem).wait()
        for g in range(CH // L):
            nvec = nbuf[pl.ds(k * CH + g * L, L)]
            for i in range(L):
                nbs = nvec[i]
                e = g * L + i
                for j in range(NEMB // L):
                    sl = pl.ds(j * L, L)
                    buf[e, sl] = buf[e, sl] * nbs
        pltpu.sync_copy(buf, acc.at[obuf.at[k]], add=True)

    @pl.loop(0, NSUPER)
    def _super(S):
        soff = base + S * SUP * CH
        pltpu.sync_copy(gidx_hbm.at[pl.ds(soff, SUP * CH)], gbuf)
        pltpu.sync_copy(oidx2_hbm.at[pl.ds(soff // CH, SUP)], obuf)
        pltpu.sync_copy(norms_hbm.at[pl.ds(soff, SUP * CH)], nbuf)
        _fire(0, rowsA, gsemA)

        @pl.loop(0, SUP // 2)
        def _pair(j):
            kA = 2 * j
            _fire(kA + 1, rowsB, gsemB)
            _process(kA, rowsA, gsemA)

            @pl.when(j < SUP // 2 - 1)
            def _pre():
                _fire(kA + 2, rowsA, gsemA)

            _process(kA + 1, rowsB, gsemB)

    plsc.subcore_barrier()
    pltpu.sync_copy(
        acc.at[pl.ds(sid * nrows, nrows)],
        out_hbm.at[cid, pl.ds(sid * nrows, nrows)],
    )

    @pl.when(sid == NS - 1)
    def _etail():
        pltpu.sync_copy(
            acc.at[pl.ds(NS * nrows, 16)],
            out_hbm.at[cid, pl.ds(NS * nrows, 16)],
        )


def _layer_call(z, gidx, oidx2, norms):
    zflat = z.reshape(NNODES * R_TOTAL, NEMB)
    return pl.kernel(
        _layer_body,
        out_type=jax.ShapeDtypeStruct((NC, NNODES, NEMB), jnp.float32),
        mesh=_MESH,
        scratch_types=[
            pltpu.VMEM((SUP * CH,), jnp.int32),
            pltpu.VMEM((SUP, CH), jnp.int32),
            pltpu.VMEM((SUP * CH,), jnp.float32),
            pltpu.VMEM((CH, NEMB), jnp.float32),
            pltpu.VMEM((CH, NEMB), jnp.float32),
            pltpu.VMEM((208, NEMB), jnp.float32),
            pltpu.VMEM_SHARED((NNODES, NEMB), jnp.float32),
            pltpu.SemaphoreType.DMA,
            pltpu.SemaphoreType.DMA,
        ],
        compiler_params=pltpu.CompilerParams(
            needs_layout_passes=False, use_tc_tiling_on_sc=False),
    )(zflat, gidx, oidx2, norms)


def _combine_body(p_ref, b_ref, rel_ref, x_ref, pen_ref):
    x_ref[...] = p_ref[0] + p_ref[1] + b_ref[...]

    @pl.when(pl.program_id(0) == 0)
    def _():
        pen_ref[...] = jnp.sum(rel_ref[...] ** 2).reshape(1, 1)


def _combine_call(p, b2, relations):
    return pl.pallas_call(
        _combine_body,
        grid=(NB,),
        in_specs=[
            pl.BlockSpec((NC, BM, NEMB), lambda i: (0, i, 0)),
            pl.BlockSpec((1, NEMB), lambda i: (0, 0)),
            pl.BlockSpec((NREL, NEMB), lambda i: (0, 0)),
        ],
        out_specs=[
            pl.BlockSpec((BM, NEMB), lambda i: (i, 0)),
            pl.BlockSpec((1, 1), lambda i: (0, 0)),
        ],
        out_shape=[
            jax.ShapeDtypeStruct((NNODES, NEMB), jnp.float32),
            jax.ShapeDtypeStruct((1, 1), jnp.float32),
        ],
    )(p, b2, relations)


def _tgather_body(x_hbm, rel_hbm, ts_hbm, tp_hbm, to_hbm, out_hbm,
                  tsv, tpv, tov, A, B, C, sem):
    cid = lax.axis_index("c")
    sid = lax.axis_index("s")
    wid = sid * NC + cid
    base = wid * T_PER_TILE

    @pl.loop(0, T_CHUNKS)
    def _chunk(t):
        off = base + t * CH
        pltpu.sync_copy(ts_hbm.at[pl.ds(off, CH)], tsv)
        pltpu.sync_copy(tp_hbm.at[pl.ds(off, CH)], tpv)
        pltpu.sync_copy(to_hbm.at[pl.ds(off, CH)], tov)
        pltpu.async_copy(x_hbm.at[tsv], A, sem).wait()
        pltpu.async_copy(x_hbm.at[tov], B, sem).wait()
        pltpu.async_copy(rel_hbm.at[tpv], C, sem).wait()
        pltpu.sync_copy(A, out_hbm.at[0, pl.ds(off, CH)])
        pltpu.sync_copy(B, out_hbm.at[1, pl.ds(off, CH)])
        pltpu.sync_copy(C, out_hbm.at[2, pl.ds(off, CH)])


def _tgather_call(x2, relations, ts, tp, to):
    return pl.kernel(
        _tgather_body,
        out_type=jax.ShapeDtypeStruct((3, N_TRIPLES, NEMB), jnp.float32),
        mesh=_MESH,
        compiler_params=pltpu.CompilerParams(use_tc_tiling_on_sc=False),
        scratch_types=[
            pltpu.VMEM((CH,), jnp.int32),
            pltpu.VMEM((CH,), jnp.int32),
            pltpu.VMEM((CH,), jnp.int32),
            pltpu.VMEM((CH, NEMB), jnp.float32),
            pltpu.VMEM((CH, NEMB), jnp.float32),
            pltpu.VMEM((CH, NEMB), jnp.float32),
            pltpu.SemaphoreType.DMA,
        ],
    )(x2, relations, ts, tp, to)


BT = 2048


def _scores_body(abc_ref, s_ref):
    prod = abc_ref[0] * abc_ref[1] * abc_ref[2]
    s_ref[...] = jnp.sum(prod, axis=-1, keepdims=True)


def _scores_call(abc):
    out = pl.pallas_call(
        _scores_body,
        grid=(N_TRIPLES // BT,),
        in_specs=[pl.BlockSpec((3, BT, NEMB), lambda i: (0, i, 0))],
        out_specs=pl.BlockSpec((BT, 1), lambda i: (i, 0)),
        out_shape=jax.ShapeDtypeStruct((N_TRIPLES, 1), jnp.float32),
    )(abc)
    return out.reshape(N_TRIPLES)


def kernel(node_embeddings, node_embeddings_bias, W1, b1, W2, b2, relations,
           graph, triples):
    s = graph[:, 0].astype(jnp.int32)
    r = (graph[:, 1] % NREL).astype(jnp.int32)
    o = graph[:, 2].astype(jnp.int32)
    loop = jnp.arange(NNODES, dtype=jnp.int32)
    s_aug = jnp.concatenate([s, o, loop])
    o_aug = jnp.concatenate([o, s, loop])
    r_aug = jnp.concatenate([r, r + NREL, jnp.full((NNODES,), 2 * NREL, jnp.int32)])

    pad = EP - E_REAL
    gidx = jnp.concatenate([s_aug * R_TOTAL + r_aug, jnp.zeros((pad,), jnp.int32)])
    seg = jnp.concatenate([r_aug * NNODES + o_aug, jnp.full((pad,), DEAD_SEG, jnp.int32)])
    oidx2 = jnp.concatenate([o_aug, jnp.zeros((pad,), jnp.int32)]).reshape(EP // CH, CH)

    ts = triples[:, 0].astype(jnp.int32)
    tp = (triples[:, 1] % NREL).astype(jnp.int32)
    to = triples[:, 2].astype(jnp.int32)

    bias = node_embeddings_bias.reshape(1, NEMB)
    b1r = b1.reshape(1, NEMB)
    b2r = b2.reshape(1, NEMB)

    counts2 = _counts_call(seg)
    inv = _inv_call(counts2)
    norms = _norms_call(seg, inv)

    W1cat = W1.transpose(1, 0, 2).reshape(NEMB, R_TOTAL * NEMB)
    W2cat = W2.transpose(1, 0, 2).reshape(NEMB, R_TOTAL * NEMB)

    z1 = _z1_call(node_embeddings, bias, W1cat)
    p1 = _layer_call(z1, gidx, oidx2, norms)

    z2 = _z2_call(p1, b1r, W2cat)
    p2 = _layer_call(z2, gidx, oidx2, norms)

    x2, pen = _combine_call(p2, b2r, relations)
    abc = _tgather_call(x2, relations, ts, tp, to)
    scores = _scores_call(abc)
    return (scores, pen.reshape(()))


# speedup vs baseline: 10.4091x; 1.1023x over previous
import functools

import jax
import jax.numpy as jnp
from jax import lax
from jax.experimental import pallas as pl
from jax.experimental.pallas import tpu as pltpu
from jax.experimental.pallas import tpu_sc as plsc

NNODES = 10000
NREL = 16
NEMB = 64
R_TOTAL = 2 * NREL + 1
SEGTOT = R_TOTAL * NNODES

NC, NS, L = 2, 16, 16
NW = NC * NS

E_REAL = 2 * 320000 + NNODES
CH = 128
SUP = 32
NCHUNK = 160
NSUPER = NCHUNK // SUP
PER_TILE = NCHUNK * CH
EP = NW * PER_TILE

SEG_FLAT = 331776
SROWS = SEG_FLAT // 16
DEAD_SEG = SEGTOT

NB = 10
BM = NNODES // NB

N_TRIPLES = 16384
T_PER_TILE = N_TRIPLES // NW
T_CHUNKS = T_PER_TILE // CH

_MESH = plsc.VectorSubcoreMesh(core_axis_name="c", subcore_axis_name="s")


def _iota16():
    return lax.iota(jnp.int32, L)


def _z1_body(x_ref, b_ref, w_ref, z_ref):
    xb = jax.nn.relu(x_ref[...] + b_ref[...])
    z_ref[...] = jnp.dot(xb, w_ref[...], preferred_element_type=jnp.float32)


def _z1_call(x, bias, Wcat):
    return pl.pallas_call(
        _z1_body,
        grid=(NB,),
        in_specs=[
            pl.BlockSpec((BM, NEMB), lambda i: (i, 0)),
            pl.BlockSpec((1, NEMB), lambda i: (0, 0)),
            pl.BlockSpec((NEMB, R_TOTAL * NEMB), lambda i: (0, 0)),
        ],
        out_specs=pl.BlockSpec((BM, R_TOTAL * NEMB), lambda i: (i, 0)),
        out_shape=jax.ShapeDtypeStruct((NNODES, R_TOTAL * NEMB), jnp.float32),
    )(x, bias, Wcat)


def _z2_body(p_ref, b_ref, w_ref, z_ref):
    xb = jax.nn.relu(p_ref[0] + p_ref[1] + b_ref[...])
    z_ref[...] = jnp.dot(xb, w_ref[...], preferred_element_type=jnp.float32)


def _z2_call(p, b1, Wcat):
    return pl.pallas_call(
        _z2_body,
        grid=(NB,),
        in_specs=[
            pl.BlockSpec((NC, BM, NEMB), lambda i: (0, i, 0)),
            pl.BlockSpec((1, NEMB), lambda i: (0, 0)),
            pl.BlockSpec((NEMB, R_TOTAL * NEMB), lambda i: (0, 0)),
        ],
        out_specs=pl.BlockSpec((BM, R_TOTAL * NEMB), lambda i: (i, 0)),
        out_shape=jax.ShapeDtypeStruct((NNODES, R_TOTAL * NEMB), jnp.float32),
    )(p, b1, Wcat)


def _counts_body(seg_hbm, out0_hbm, out1_hbm, segv, onesv, zbuf, acc, sem):
    del sem
    cid = lax.axis_index("c")
    sid = lax.axis_index("s")
    wid = sid * NC + cid
    base = wid * PER_TILE
    seg_per_tile = SEG_FLAT // NS
    zsize = seg_per_tile // 8

    @pl.loop(0, zsize // L)
    def _zero(i):
        zbuf[pl.ds(i * L, L)] = jnp.zeros((L,), jnp.float32)

    @pl.loop(0, 8)
    def _zacc(k):
        pltpu.sync_copy(zbuf, acc.at[pl.ds(sid * seg_per_tile + k * zsize, zsize)])

    @pl.loop(0, CH // L)
    def _ones(g):
        onesv[pl.ds(g * L, L)] = jnp.ones((L,), jnp.float32)

    plsc.subcore_barrier()

    @pl.loop(0, NCHUNK)
    def _chunk(t):
        off = base + t * CH
        pltpu.sync_copy(seg_hbm.at[pl.ds(off, CH)], segv)
        pltpu.sync_copy(onesv, acc.at[segv], add=True)

    plsc.subcore_barrier()

    @pl.when(cid == 0)
    def _exp0():
        pltpu.sync_copy(
            acc.at[pl.ds(sid * seg_per_tile, seg_per_tile)],
            out0_hbm.at[pl.ds(sid * seg_per_tile, seg_per_tile)],
        )

    @pl.when(cid == 1)
    def _exp1():
        pltpu.sync_copy(
            acc.at[pl.ds(sid * seg_per_tile, seg_per_tile)],
            out1_hbm.at[pl.ds(sid * seg_per_tile, seg_per_tile)],
        )


def _counts_call(seg):
    return pl.kernel(
        _counts_body,
        out_type=[jax.ShapeDtypeStruct((SEG_FLAT,), jnp.float32),
                  jax.ShapeDtypeStruct((SEG_FLAT,), jnp.float32)],
        mesh=_MESH,
        scratch_types=[
            pltpu.VMEM((CH,), jnp.int32),
            pltpu.VMEM((CH,), jnp.float32),
            pltpu.VMEM((SEG_FLAT // NS // 8,), jnp.float32),
            pltpu.VMEM_SHARED((SEG_FLAT,), jnp.float32),
            pltpu.SemaphoreType.DMA,
        ],
    )(seg)


def _norms_body(seg_hbm, cnt0_hbm, cnt1_hbm, out_hbm,
                sbuf, c0A, c1A, c0B, c1B, nwb, semA, semB):
    cid = lax.axis_index("c")
    sid = lax.axis_index("s")
    wid = sid * NC + cid
    base = wid * PER_TILE

    def _fire(k, b0, b1, sem):
        idx = sbuf.at[pl.ds(k * CH, CH)]
        pltpu.async_copy(cnt0_hbm.at[idx], b0, sem)
        pltpu.async_copy(cnt1_hbm.at[idx], b1, sem)

    def _process(k, b0, b1, sem, soff):
        idx = sbuf.at[pl.ds(k * CH, CH)]
        pltpu.make_async_copy(cnt0_hbm.at[idx], b0, sem).wait()
        pltpu.make_async_copy(cnt1_hbm.at[idx], b1, sem).wait()
        for g in range(CH // L):
            sl = pl.ds(g * L, L)
            sg = sbuf[pl.ds(k * CH + g * L, L)]
            c = b0[sl] + b1[sl]
            inv = 1.0 / jnp.maximum(c, 1.0)
            nwb[sl] = jnp.where(sg < SEGTOT, inv, 0.0)
        pltpu.sync_copy(nwb, out_hbm.at[pl.ds(soff + k * CH, CH)])

    @pl.loop(0, NSUPER)
    def _super(S):
        soff = base + S * SUP * CH
        pltpu.sync_copy(seg_hbm.at[pl.ds(soff, SUP * CH)], sbuf)
        _fire(0, c0A, c1A, semA)

        @pl.loop(0, SUP // 2)
        def _pair(j):
            kA = 2 * j
            _fire(kA + 1, c0B, c1B, semB)
            _process(kA, c0A, c1A, semA, soff)

            @pl.when(j < SUP // 2 - 1)
            def _pre():
                _fire(kA + 2, c0A, c1A, semA)

            _process(kA + 1, c0B, c1B, semB, soff)


def _norms_call(seg, cnt0, cnt1):
    return pl.kernel(
        _norms_body,
        out_type=jax.ShapeDtypeStruct((EP,), jnp.float32),
        mesh=_MESH,
        scratch_types=[
            pltpu.VMEM((SUP * CH,), jnp.int32),
            pltpu.VMEM((CH,), jnp.float32),
            pltpu.VMEM((CH,), jnp.float32),
            pltpu.VMEM((CH,), jnp.float32),
            pltpu.VMEM((CH,), jnp.float32),
            pltpu.VMEM((CH,), jnp.float32),
            pltpu.SemaphoreType.DMA,
            pltpu.SemaphoreType.DMA,
        ],
    )(seg, cnt0, cnt1)


def _layer_body(z_hbm, gidx_hbm, oidx2_hbm, norms_hbm, out_hbm,
                gbuf, obuf, nbuf, rowsA, rowsB, zbuf, acc, gsemA, gsemB):
    cid = lax.axis_index("c")
    sid = lax.axis_index("s")
    wid = sid * NC + cid
    base = wid * PER_TILE
    nrows = 624
    zrows = 208

    @pl.loop(0, zrows)
    def _zero(i):
        for j in range(NEMB // L):
            zbuf[i, pl.ds(j * L, L)] = jnp.zeros((L,), jnp.float32)

    @pl.loop(0, nrows // zrows)
    def _zacc(k):
        pltpu.sync_copy(zbuf, acc.at[pl.ds(sid * nrows + k * zrows, zrows)])

    @pl.when(sid == NS - 1)
    def _ztail():
        pltpu.sync_copy(zbuf.at[pl.ds(0, 16)], acc.at[pl.ds(NS * nrows, 16)])

    plsc.subcore_barrier()

    def _fire(k, buf, sem):
        pltpu.async_copy(z_hbm.at[gbuf.at[pl.ds(k * CH, CH)]], buf, sem)

    def _process(k, buf, sem):
        pltpu.make_async_copy(
            z_hbm.at[gbuf.at[pl.ds(k * CH, CH)]], buf, sem).wait()
        for g in range(CH // L):
            nvec = nbuf[pl.ds(k * CH + g * L, L)]
            for i in range(L):
                nbs = nvec[i]
                e = g * L + i
                for j in range(NEMB // L):
                    sl = pl.ds(j * L, L)
                    buf[e, sl] = buf[e, sl] * nbs
        pltpu.sync_copy(buf, acc.at[obuf.at[k]], add=True)

    @pl.loop(0, NSUPER)
    def _super(S):
        soff = base + S * SUP * CH
        pltpu.sync_copy(gidx_hbm.at[pl.ds(soff, SUP * CH)], gbuf)
        pltpu.sync_copy(oidx2_hbm.at[pl.ds(soff // CH, SUP)], obuf)
        pltpu.sync_copy(norms_hbm.at[pl.ds(soff, SUP * CH)], nbuf)
        _fire(0, rowsA, gsemA)

        @pl.loop(0, SUP // 2)
        def _pair(j):
            kA = 2 * j
            _fire(kA + 1, rowsB, gsemB)
            _process(kA, rowsA, gsemA)

            @pl.when(j < SUP // 2 - 1)
            def _pre():
                _fire(kA + 2, rowsA, gsemA)

            _process(kA + 1, rowsB, gsemB)

    plsc.subcore_barrier()
    pltpu.sync_copy(
        acc.at[pl.ds(sid * nrows, nrows)],
        out_hbm.at[cid, pl.ds(sid * nrows, nrows)],
    )

    @pl.when(sid == NS - 1)
    def _etail():
        pltpu.sync_copy(
            acc.at[pl.ds(NS * nrows, 16)],
            out_hbm.at[cid, pl.ds(NS * nrows, 16)],
        )


def _layer_call(z, gidx, oidx2, norms):
    zflat = z.reshape(NNODES * R_TOTAL, NEMB)
    return pl.kernel(
        _layer_body,
        out_type=jax.ShapeDtypeStruct((NC, NNODES, NEMB), jnp.float32),
        mesh=_MESH,
        scratch_types=[
            pltpu.VMEM((SUP * CH,), jnp.int32),
            pltpu.VMEM((SUP, CH), jnp.int32),
            pltpu.VMEM((SUP * CH,), jnp.float32),
            pltpu.VMEM((CH, NEMB), jnp.float32),
            pltpu.VMEM((CH, NEMB), jnp.float32),
            pltpu.VMEM((208, NEMB), jnp.float32),
            pltpu.VMEM_SHARED((NNODES, NEMB), jnp.float32),
            pltpu.SemaphoreType.DMA,
            pltpu.SemaphoreType.DMA,
        ],
        compiler_params=pltpu.CompilerParams(
            needs_layout_passes=False, use_tc_tiling_on_sc=False),
    )(zflat, gidx, oidx2, norms)


def _combine_body(p_ref, b_ref, rel_ref, x_ref, pen_ref):
    x_ref[...] = p_ref[0] + p_ref[1] + b_ref[...]

    @pl.when(pl.program_id(0) == 0)
    def _():
        pen_ref[...] = jnp.sum(rel_ref[...] ** 2).reshape(1, 1)


def _combine_call(p, b2, relations):
    return pl.pallas_call(
        _combine_body,
        grid=(NB,),
        in_specs=[
            pl.BlockSpec((NC, BM, NEMB), lambda i: (0, i, 0)),
            pl.BlockSpec((1, NEMB), lambda i: (0, 0)),
            pl.BlockSpec((NREL, NEMB), lambda i: (0, 0)),
        ],
        out_specs=[
            pl.BlockSpec((BM, NEMB), lambda i: (i, 0)),
            pl.BlockSpec((1, 1), lambda i: (0, 0)),
        ],
        out_shape=[
            jax.ShapeDtypeStruct((NNODES, NEMB), jnp.float32),
            jax.ShapeDtypeStruct((1, 1), jnp.float32),
        ],
    )(p, b2, relations)


def _tgather_body(x_hbm, rel_hbm, ts_hbm, tp_hbm, to_hbm, out_hbm,
                  tsb, tpb, tob, A0, B0, C0, A1, B1, C1, sem0, sem1):
    cid = lax.axis_index("c")
    sid = lax.axis_index("s")
    wid = sid * NC + cid
    base = wid * T_PER_TILE

    pltpu.sync_copy(ts_hbm.at[pl.ds(base, T_PER_TILE)], tsb)
    pltpu.sync_copy(tp_hbm.at[pl.ds(base, T_PER_TILE)], tpb)
    pltpu.sync_copy(to_hbm.at[pl.ds(base, T_PER_TILE)], tob)

    def _fire(k, A, B, C, sem):
        sl = pl.ds(k * CH, CH)
        pltpu.async_copy(x_hbm.at[tsb.at[sl]], A, sem)
        pltpu.async_copy(x_hbm.at[tob.at[sl]], B, sem)
        pltpu.async_copy(rel_hbm.at[tpb.at[sl]], C, sem)

    def _proc(k, A, B, C, sem):
        sl = pl.ds(k * CH, CH)
        pltpu.make_async_copy(x_hbm.at[tsb.at[sl]], A, sem).wait()
        pltpu.make_async_copy(x_hbm.at[tob.at[sl]], B, sem).wait()
        pltpu.make_async_copy(rel_hbm.at[tpb.at[sl]], C, sem).wait()
        off = base + k * CH
        pltpu.sync_copy(A, out_hbm.at[0, pl.ds(off, CH)])
        pltpu.sync_copy(B, out_hbm.at[1, pl.ds(off, CH)])
        pltpu.sync_copy(C, out_hbm.at[2, pl.ds(off, CH)])

    _fire(0, A0, B0, C0, sem0)
    _fire(1, A1, B1, C1, sem1)
    _proc(0, A0, B0, C0, sem0)
    _fire(2, A0, B0, C0, sem0)
    _proc(1, A1, B1, C1, sem1)
    _fire(3, A1, B1, C1, sem1)
    _proc(2, A0, B0, C0, sem0)
    _proc(3, A1, B1, C1, sem1)


def _tgather_call(x2, relations, ts, tp, to):
    return pl.kernel(
        _tgather_body,
        out_type=jax.ShapeDtypeStruct((3, N_TRIPLES, NEMB), jnp.float32),
        mesh=_MESH,
        compiler_params=pltpu.CompilerParams(use_tc_tiling_on_sc=False),
        scratch_types=[
            pltpu.VMEM((T_PER_TILE,), jnp.int32),
            pltpu.VMEM((T_PER_TILE,), jnp.int32),
            pltpu.VMEM((T_PER_TILE,), jnp.int32),
            pltpu.VMEM((CH, NEMB), jnp.float32),
            pltpu.VMEM((CH, NEMB), jnp.float32),
            pltpu.VMEM((CH, NEMB), jnp.float32),
            pltpu.VMEM((CH, NEMB), jnp.float32),
            pltpu.VMEM((CH, NEMB), jnp.float32),
            pltpu.VMEM((CH, NEMB), jnp.float32),
            pltpu.SemaphoreType.DMA,
            pltpu.SemaphoreType.DMA,
        ],
    )(x2, relations, ts, tp, to)


BT = 2048


def _scores_body(abc_ref, s_ref):
    prod = abc_ref[0] * abc_ref[1] * abc_ref[2]
    s_ref[...] = jnp.sum(prod, axis=-1, keepdims=True)


def _scores_call(abc):
    out = pl.pallas_call(
        _scores_body,
        grid=(N_TRIPLES // BT,),
        in_specs=[pl.BlockSpec((3, BT, NEMB), lambda i: (0, i, 0))],
        out_specs=pl.BlockSpec((BT, 1), lambda i: (i, 0)),
        out_shape=jax.ShapeDtypeStruct((N_TRIPLES, 1), jnp.float32),
    )(abc)
    return out.reshape(N_TRIPLES)


def kernel(node_embeddings, node_embeddings_bias, W1, b1, W2, b2, relations,
           graph, triples):
    s = graph[:, 0].astype(jnp.int32)
    r = (graph[:, 1] % NREL).astype(jnp.int32)
    o = graph[:, 2].astype(jnp.int32)
    loop = jnp.arange(NNODES, dtype=jnp.int32)
    s_aug = jnp.concatenate([s, o, loop])
    o_aug = jnp.concatenate([o, s, loop])
    r_aug = jnp.concatenate([r, r + NREL, jnp.full((NNODES,), 2 * NREL, jnp.int32)])

    pad = EP - E_REAL
    gidx = jnp.concatenate([s_aug * R_TOTAL + r_aug, jnp.zeros((pad,), jnp.int32)])
    seg = jnp.concatenate([r_aug * NNODES + o_aug, jnp.full((pad,), DEAD_SEG, jnp.int32)])
    oidx2 = jnp.concatenate([o_aug, jnp.zeros((pad,), jnp.int32)]).reshape(EP // CH, CH)

    ts = triples[:, 0].astype(jnp.int32)
    tp = (triples[:, 1] % NREL).astype(jnp.int32)
    to = triples[:, 2].astype(jnp.int32)

    bias = node_embeddings_bias.reshape(1, NEMB)
    b1r = b1.reshape(1, NEMB)
    b2r = b2.reshape(1, NEMB)

    cnt0, cnt1 = _counts_call(seg)
    norms = _norms_call(seg, cnt0, cnt1)

    W1cat = W1.transpose(1, 0, 2).reshape(NEMB, R_TOTAL * NEMB)
    W2cat = W2.transpose(1, 0, 2).reshape(NEMB, R_TOTAL * NEMB)

    z1 = _z1_call(node_embeddings, bias, W1cat)
    p1 = _layer_call(z1, gidx, oidx2, norms)

    z2 = _z2_call(p1, b1r, W2cat)
    p2 = _layer_call(z2, gidx, oidx2, norms)

    x2, pen = _combine_call(p2, b2r, relations)
    abc = _tgather_call(x2, relations, ts, tp, to)
    scores = _scores_call(abc)
    return (scores, pen.reshape(()))


# speedup vs baseline: 10.8545x; 1.0428x over previous
import functools

import jax
import jax.numpy as jnp
from jax import lax
from jax.experimental import pallas as pl
from jax.experimental.pallas import tpu as pltpu
from jax.experimental.pallas import tpu_sc as plsc

NNODES = 10000
NREL = 16
NEMB = 64
R_TOTAL = 2 * NREL + 1
SEGTOT = R_TOTAL * NNODES

NC, NS, L = 2, 16, 16
NW = NC * NS

E_REAL = 2 * 320000 + NNODES
CH = 128
SUP = 32
NCHUNK = 160
NSUPER = NCHUNK // SUP
PER_TILE = NCHUNK * CH
EP = NW * PER_TILE

SEG_FLAT = 331776
SROWS = SEG_FLAT // 16
DEAD_SEG = SEGTOT

NB = 10
BM = NNODES // NB

N_TRIPLES = 16384
T_PER_TILE = N_TRIPLES // NW
T_CHUNKS = T_PER_TILE // CH

_MESH = plsc.VectorSubcoreMesh(core_axis_name="c", subcore_axis_name="s")


def _iota16():
    return lax.iota(jnp.int32, L)


def _z1_body(x_ref, b_ref, w_ref, z_ref):
    xb = jax.nn.relu(x_ref[...] + b_ref[...])
    z_ref[...] = jnp.dot(xb, w_ref[...], preferred_element_type=jnp.float32)


def _z1_call(x, bias, Wcat):
    return pl.pallas_call(
        _z1_body,
        grid=(NB,),
        in_specs=[
            pl.BlockSpec((BM, NEMB), lambda i: (i, 0)),
            pl.BlockSpec((1, NEMB), lambda i: (0, 0)),
            pl.BlockSpec((NEMB, R_TOTAL * NEMB), lambda i: (0, 0)),
        ],
        out_specs=pl.BlockSpec((BM, R_TOTAL * NEMB), lambda i: (i, 0)),
        out_shape=jax.ShapeDtypeStruct((NNODES, R_TOTAL * NEMB), jnp.float32),
    )(x, bias, Wcat)


def _z2_body(p_ref, b_ref, w_ref, z_ref):
    xb = jax.nn.relu(p_ref[0] + p_ref[1] + b_ref[...])
    z_ref[...] = jnp.dot(xb, w_ref[...], preferred_element_type=jnp.float32)


def _z2_call(p, b1, Wcat):
    return pl.pallas_call(
        _z2_body,
        grid=(NB,),
        in_specs=[
            pl.BlockSpec((NC, BM, NEMB), lambda i: (0, i, 0)),
            pl.BlockSpec((1, NEMB), lambda i: (0, 0)),
            pl.BlockSpec((NEMB, R_TOTAL * NEMB), lambda i: (0, 0)),
        ],
        out_specs=pl.BlockSpec((BM, R_TOTAL * NEMB), lambda i: (i, 0)),
        out_shape=jax.ShapeDtypeStruct((NNODES, R_TOTAL * NEMB), jnp.float32),
    )(p, b1, Wcat)


def _counts_body(seg_hbm, out0_hbm, out1_hbm, segA, segB, onesv, zbuf, acc, semA, semB):
    cid = lax.axis_index("c")
    sid = lax.axis_index("s")
    wid = sid * NC + cid
    base = wid * PER_TILE
    seg_per_tile = SEG_FLAT // NS
    zsize = seg_per_tile // 8

    @pl.loop(0, zsize // L)
    def _zero(i):
        zbuf[pl.ds(i * L, L)] = jnp.zeros((L,), jnp.float32)

    @pl.loop(0, 8)
    def _zacc(k):
        pltpu.sync_copy(zbuf, acc.at[pl.ds(sid * seg_per_tile + k * zsize, zsize)])

    @pl.loop(0, CH // L)
    def _ones(g):
        onesv[pl.ds(g * L, L)] = jnp.ones((L,), jnp.float32)

    plsc.subcore_barrier()

    def _stage(t, buf, sem):
        pltpu.async_copy(seg_hbm.at[pl.ds(base + t * CH, CH)], buf, sem)

    def _proc(t, buf, sem):
        pltpu.make_async_copy(
            seg_hbm.at[pl.ds(base + t * CH, CH)], buf, sem).wait()
        pltpu.sync_copy(onesv, acc.at[buf], add=True)

    _stage(0, segA, semA)

    @pl.loop(0, NCHUNK // 2)
    def _pair(t):
        c = 2 * t
        _stage(c + 1, segB, semB)
        _proc(c, segA, semA)

        @pl.when(t < NCHUNK // 2 - 1)
        def _pre():
            _stage(c + 2, segA, semA)

        _proc(c + 1, segB, semB)

    plsc.subcore_barrier()

    @pl.when(cid == 0)
    def _exp0():
        pltpu.sync_copy(
            acc.at[pl.ds(sid * seg_per_tile, seg_per_tile)],
            out0_hbm.at[pl.ds(sid * seg_per_tile, seg_per_tile)],
        )

    @pl.when(cid == 1)
    def _exp1():
        pltpu.sync_copy(
            acc.at[pl.ds(sid * seg_per_tile, seg_per_tile)],
            out1_hbm.at[pl.ds(sid * seg_per_tile, seg_per_tile)],
        )


def _counts_call(seg):
    return pl.kernel(
        _counts_body,
        out_type=[jax.ShapeDtypeStruct((SEG_FLAT,), jnp.float32),
                  jax.ShapeDtypeStruct((SEG_FLAT,), jnp.float32)],
        mesh=_MESH,
        scratch_types=[
            pltpu.VMEM((CH,), jnp.int32),
            pltpu.VMEM((CH,), jnp.int32),
            pltpu.VMEM((CH,), jnp.float32),
            pltpu.VMEM((SEG_FLAT // NS // 8,), jnp.float32),
            pltpu.VMEM_SHARED((SEG_FLAT,), jnp.float32),
            pltpu.SemaphoreType.DMA,
            pltpu.SemaphoreType.DMA,
        ],
    )(seg)


def _norms_body(seg_hbm, cnt0_hbm, cnt1_hbm, out_hbm,
                sbuf, c0A, c1A, c0B, c1B, nwb, semA, semB):
    cid = lax.axis_index("c")
    sid = lax.axis_index("s")
    wid = sid * NC + cid
    base = wid * PER_TILE

    def _fire(k, b0, b1, sem):
        idx = sbuf.at[pl.ds(k * CH, CH)]
        pltpu.async_copy(cnt0_hbm.at[idx], b0, sem)
        pltpu.async_copy(cnt1_hbm.at[idx], b1, sem)

    def _process(k, b0, b1, sem, soff):
        idx = sbuf.at[pl.ds(k * CH, CH)]
        pltpu.make_async_copy(cnt0_hbm.at[idx], b0, sem).wait()
        pltpu.make_async_copy(cnt1_hbm.at[idx], b1, sem).wait()
        for g in range(CH // L):
            sl = pl.ds(g * L, L)
            sg = sbuf[pl.ds(k * CH + g * L, L)]
            c = b0[sl] + b1[sl]
            inv = 1.0 / jnp.maximum(c, 1.0)
            nwb[sl] = jnp.where(sg < SEGTOT, inv, 0.0)
        pltpu.sync_copy(nwb, out_hbm.at[pl.ds(soff + k * CH, CH)])

    @pl.loop(0, NSUPER)
    def _super(S):
        soff = base + S * SUP * CH
        pltpu.sync_copy(seg_hbm.at[pl.ds(soff, SUP * CH)], sbuf)
        _fire(0, c0A, c1A, semA)

        @pl.loop(0, SUP // 2)
        def _pair(j):
            kA = 2 * j
            _fire(kA + 1, c0B, c1B, semB)
            _process(kA, c0A, c1A, semA, soff)

            @pl.when(j < SUP // 2 - 1)
            def _pre():
                _fire(kA + 2, c0A, c1A, semA)

            _process(kA + 1, c0B, c1B, semB, soff)


def _norms_call(seg, cnt0, cnt1):
    return pl.kernel(
        _norms_body,
        out_type=jax.ShapeDtypeStruct((EP,), jnp.float32),
        mesh=_MESH,
        scratch_types=[
            pltpu.VMEM((SUP * CH,), jnp.int32),
            pltpu.VMEM((CH,), jnp.float32),
            pltpu.VMEM((CH,), jnp.float32),
            pltpu.VMEM((CH,), jnp.float32),
            pltpu.VMEM((CH,), jnp.float32),
            pltpu.VMEM((CH,), jnp.float32),
            pltpu.SemaphoreType.DMA,
            pltpu.SemaphoreType.DMA,
        ],
    )(seg, cnt0, cnt1)


def _layer_body(z_hbm, gidx_hbm, oidx2_hbm, norms_hbm, out_hbm,
                gbuf, obuf, nbuf, rowsA, rowsB, zbuf, acc, gsemA, gsemB):
    cid = lax.axis_index("c")
    sid = lax.axis_index("s")
    wid = sid * NC + cid
    base = wid * PER_TILE
    nrows = 624
    zrows = 208

    @pl.loop(0, zrows)
    def _zero(i):
        for j in range(NEMB // L):
            zbuf[i, pl.ds(j * L, L)] = jnp.zeros((L,), jnp.float32)

    @pl.loop(0, nrows // zrows)
    def _zacc(k):
        pltpu.sync_copy(zbuf, acc.at[pl.ds(sid * nrows + k * zrows, zrows)])

    @pl.when(sid == NS - 1)
    def _ztail():
        pltpu.sync_copy(zbuf.at[pl.ds(0, 16)], acc.at[pl.ds(NS * nrows, 16)])

    plsc.subcore_barrier()

    def _fire(k, buf, sem):
        pltpu.async_copy(z_hbm.at[gbuf.at[pl.ds(k * CH, CH)]], buf, sem)

    def _process(k, buf, sem):
        pltpu.make_async_copy(
            z_hbm.at[gbuf.at[pl.ds(k * CH, CH)]], buf, sem).wait()
        for g in range(CH // L):
            nvec = nbuf[pl.ds(k * CH + g * L, L)]
            for i in range(L):
                nbs = nvec[i]
                e = g * L + i
                for j in range(NEMB // L):
                    sl = pl.ds(j * L, L)
                    buf[e, sl] = buf[e, sl] * nbs
        pltpu.sync_copy(buf, acc.at[obuf.at[k]], add=True)

    @pl.loop(0, NSUPER)
    def _super(S):
        soff = base + S * SUP * CH
        pltpu.sync_copy(gidx_hbm.at[pl.ds(soff, SUP * CH)], gbuf)
        pltpu.sync_copy(oidx2_hbm.at[pl.ds(soff // CH, SUP)], obuf)
        pltpu.sync_copy(norms_hbm.at[pl.ds(soff, SUP * CH)], nbuf)
        _fire(0, rowsA, gsemA)

        @pl.loop(0, SUP // 2)
        def _pair(j):
            kA = 2 * j
            _fire(kA + 1, rowsB, gsemB)
            _process(kA, rowsA, gsemA)

            @pl.when(j < SUP // 2 - 1)
            def _pre():
                _fire(kA + 2, rowsA, gsemA)

            _process(kA + 1, rowsB, gsemB)

    plsc.subcore_barrier()
    pltpu.sync_copy(
        acc.at[pl.ds(sid * nrows, nrows)],
        out_hbm.at[cid, pl.ds(sid * nrows, nrows)],
    )

    @pl.when(sid == NS - 1)
    def _etail():
        pltpu.sync_copy(
            acc.at[pl.ds(NS * nrows, 16)],
            out_hbm.at[cid, pl.ds(NS * nrows, 16)],
        )


def _layer_call(z, gidx, oidx2, norms):
    zflat = z.reshape(NNODES * R_TOTAL, NEMB)
    return pl.kernel(
        _layer_body,
        out_type=jax.ShapeDtypeStruct((NC, NNODES, NEMB), jnp.float32),
        mesh=_MESH,
        scratch_types=[
            pltpu.VMEM((SUP * CH,), jnp.int32),
            pltpu.VMEM((SUP, CH), jnp.int32),
            pltpu.VMEM((SUP * CH,), jnp.float32),
            pltpu.VMEM((CH, NEMB), jnp.float32),
            pltpu.VMEM((CH, NEMB), jnp.float32),
            pltpu.VMEM((208, NEMB), jnp.float32),
            pltpu.VMEM_SHARED((NNODES, NEMB), jnp.float32),
            pltpu.SemaphoreType.DMA,
            pltpu.SemaphoreType.DMA,
        ],
        compiler_params=pltpu.CompilerParams(
            needs_layout_passes=False, use_tc_tiling_on_sc=False),
    )(zflat, gidx, oidx2, norms)


def _combine_body(p_ref, b_ref, rel_ref, x_ref, pen_ref):
    x_ref[...] = p_ref[0] + p_ref[1] + b_ref[...]

    @pl.when(pl.program_id(0) == 0)
    def _():
        pen_ref[...] = jnp.sum(rel_ref[...] ** 2).reshape(1, 1)


def _combine_call(p, b2, relations):
    return pl.pallas_call(
        _combine_body,
        grid=(NB,),
        in_specs=[
            pl.BlockSpec((NC, BM, NEMB), lambda i: (0, i, 0)),
            pl.BlockSpec((1, NEMB), lambda i: (0, 0)),
            pl.BlockSpec((NREL, NEMB), lambda i: (0, 0)),
        ],
        out_specs=[
            pl.BlockSpec((BM, NEMB), lambda i: (i, 0)),
            pl.BlockSpec((1, 1), lambda i: (0, 0)),
        ],
        out_shape=[
            jax.ShapeDtypeStruct((NNODES, NEMB), jnp.float32),
            jax.ShapeDtypeStruct((1, 1), jnp.float32),
        ],
    )(p, b2, relations)


def _tgather_body(x_hbm, rel_hbm, ts_hbm, tp_hbm, to_hbm, out_hbm,
                  tsb, tpb, tob, A0, B0, C0, A1, B1, C1, sem0, sem1):
    cid = lax.axis_index("c")
    sid = lax.axis_index("s")
    wid = sid * NC + cid
    base = wid * T_PER_TILE

    pltpu.sync_copy(ts_hbm.at[pl.ds(base, T_PER_TILE)], tsb)
    pltpu.sync_copy(tp_hbm.at[pl.ds(base, T_PER_TILE)], tpb)
    pltpu.sync_copy(to_hbm.at[pl.ds(base, T_PER_TILE)], tob)

    def _fire(k, A, B, C, sem):
        sl = pl.ds(k * CH, CH)
        pltpu.async_copy(x_hbm.at[tsb.at[sl]], A, sem)
        pltpu.async_copy(x_hbm.at[tob.at[sl]], B, sem)
        pltpu.async_copy(rel_hbm.at[tpb.at[sl]], C, sem)

    def _proc(k, A, B, C, sem):
        sl = pl.ds(k * CH, CH)
        pltpu.make_async_copy(x_hbm.at[tsb.at[sl]], A, sem).wait()
        pltpu.make_async_copy(x_hbm.at[tob.at[sl]], B, sem).wait()
        pltpu.make_async_copy(rel_hbm.at[tpb.at[sl]], C, sem).wait()
        off = base + k * CH
        pltpu.sync_copy(A, out_hbm.at[0, pl.ds(off, CH)])
        pltpu.sync_copy(B, out_hbm.at[1, pl.ds(off, CH)])
        pltpu.sync_copy(C, out_hbm.at[2, pl.ds(off, CH)])

    _fire(0, A0, B0, C0, sem0)
    _fire(1, A1, B1, C1, sem1)
    _proc(0, A0, B0, C0, sem0)
    _fire(2, A0, B0, C0, sem0)
    _proc(1, A1, B1, C1, sem1)
    _fire(3, A1, B1, C1, sem1)
    _proc(2, A0, B0, C0, sem0)
    _proc(3, A1, B1, C1, sem1)


def _tgather_call(x2, relations, ts, tp, to):
    return pl.kernel(
        _tgather_body,
        out_type=jax.ShapeDtypeStruct((3, N_TRIPLES, NEMB), jnp.float32),
        mesh=_MESH,
        compiler_params=pltpu.CompilerParams(use_tc_tiling_on_sc=False),
        scratch_types=[
            pltpu.VMEM((T_PER_TILE,), jnp.int32),
            pltpu.VMEM((T_PER_TILE,), jnp.int32),
            pltpu.VMEM((T_PER_TILE,), jnp.int32),
            pltpu.VMEM((CH, NEMB), jnp.float32),
            pltpu.VMEM((CH, NEMB), jnp.float32),
            pltpu.VMEM((CH, NEMB), jnp.float32),
            pltpu.VMEM((CH, NEMB), jnp.float32),
            pltpu.VMEM((CH, NEMB), jnp.float32),
            pltpu.VMEM((CH, NEMB), jnp.float32),
            pltpu.SemaphoreType.DMA,
            pltpu.SemaphoreType.DMA,
        ],
    )(x2, relations, ts, tp, to)


BT = 2048


def _scores_body(abc_ref, s_ref):
    prod = abc_ref[0] * abc_ref[1] * abc_ref[2]
    s_ref[...] = jnp.sum(prod, axis=-1, keepdims=True)


def _scores_call(abc):
    out = pl.pallas_call(
        _scores_body,
        grid=(N_TRIPLES // BT,),
        in_specs=[pl.BlockSpec((3, BT, NEMB), lambda i: (0, i, 0))],
        out_specs=pl.BlockSpec((BT, 1), lambda i: (i, 0)),
        out_shape=jax.ShapeDtypeStruct((N_TRIPLES, 1), jnp.float32),
    )(abc)
    return out.reshape(N_TRIPLES)


def kernel(node_embeddings, node_embeddings_bias, W1, b1, W2, b2, relations,
           graph, triples):
    s = graph[:, 0].astype(jnp.int32)
    r = (graph[:, 1] % NREL).astype(jnp.int32)
    o = graph[:, 2].astype(jnp.int32)
    loop = jnp.arange(NNODES, dtype=jnp.int32)
    s_aug = jnp.concatenate([s, o, loop])
    o_aug = jnp.concatenate([o, s, loop])
    r_aug = jnp.concatenate([r, r + NREL, jnp.full((NNODES,), 2 * NREL, jnp.int32)])

    pad = EP - E_REAL
    gidx = jnp.concatenate([s_aug * R_TOTAL + r_aug, jnp.zeros((pad,), jnp.int32)])
    seg = jnp.concatenate([r_aug * NNODES + o_aug, jnp.full((pad,), DEAD_SEG, jnp.int32)])
    oidx2 = jnp.concatenate([o_aug, jnp.zeros((pad,), jnp.int32)]).reshape(EP // CH, CH)

    ts = triples[:, 0].astype(jnp.int32)
    tp = (triples[:, 1] % NREL).astype(jnp.int32)
    to = triples[:, 2].astype(jnp.int32)

    bias = node_embeddings_bias.reshape(1, NEMB)
    b1r = b1.reshape(1, NEMB)
    b2r = b2.reshape(1, NEMB)

    cnt0, cnt1 = _counts_call(seg)
    norms = _norms_call(seg, cnt0, cnt1)

    W1cat = W1.transpose(1, 0, 2).reshape(NEMB, R_TOTAL * NEMB)
    W2cat = W2.transpose(1, 0, 2).reshape(NEMB, R_TOTAL * NEMB)

    z1 = _z1_call(node_embeddings, bias, W1cat)
    p1 = _layer_call(z1, gidx, oidx2, norms)

    z2 = _z2_call(p1, b1r, W2cat)
    p2 = _layer_call(z2, gidx, oidx2, norms)

    x2, pen = _combine_call(p2, b2r, relations)
    abc = _tgather_call(x2, relations, ts, tp, to)
    scores = _scores_call(abc)
    return (scores, pen.reshape(()))
